# baseline (speedup 1.0000x reference)
import jax
import jax.numpy as jnp
from jax.experimental import pallas as pl

N_NODES = 10000
N_EDGES = 160000
HID = 512
NHID = 1024
NOUT = 128
N_GRAPHS = 64
K1 = 5000
K2 = 2500


def _head_body(g_ref, w1_ref, b1_ref, w2_ref, b2_ref, o_ref):
    hdn = jnp.dot(g_ref[...], w1_ref[...], preferred_element_type=jnp.float32)
    hdn = jnp.maximum(hdn + b1_ref[...], 0.0)
    o_ref[...] = jnp.dot(hdn, w2_ref[...], preferred_element_type=jnp.float32) + b2_ref[...]


def _head(g, Wm1, bm1, Wm2, bm2):
    return pl.pallas_call(
        _head_body,
        out_shape=jax.ShapeDtypeStruct((N_GRAPHS, NOUT), jnp.float32),
    )(g, Wm1, bm1.reshape(1, NHID), Wm2, bm2.reshape(1, NOUT))


def _gcn(x, src, dst, w, W, b, n):
    h = x @ W
    deg = jax.ops.segment_sum(w, dst, num_segments=n) + 2.0
    dinv = jax.lax.rsqrt(deg)
    norm = dinv[src] * dinv[dst] * w
    agg = jax.ops.segment_sum(h[src] * norm[:, None], dst, num_segments=n)
    agg = agg + h * (2.0 * dinv * dinv)[:, None]
    return agg + b


def _topk_pool(x, src, dst, w, p, k, n):
    score = jnp.tanh((x @ p) / jnp.linalg.norm(p))
    vals, perm = jax.lax.top_k(score, k)
    x_new = x[perm] * vals[:, None]
    mapping = jnp.full((n,), -1, dtype=jnp.int32).at[perm].set(jnp.arange(k, dtype=jnp.int32))
    ns = mapping[src]
    nd = mapping[dst]
    keep = (ns >= 0) & (nd >= 0)
    w_new = jnp.where(keep, w, 0.0)
    ns = jnp.where(keep, ns, 0)
    nd = jnp.where(keep, nd, 0)
    return x_new, ns, nd, w_new, perm


def kernel(x, edge_index, batch, Wd0, bd0, p1, Wd1, bd1, p2, Wd2, bd2, Wu0, bu0, Wu1, bu1, Wm1, bm1, Wm2, bm2):
    src = edge_index[0].astype(jnp.int32)
    dst = edge_index[1].astype(jnp.int32)
    w0 = jnp.ones((N_EDGES,), dtype=jnp.float32)
    h0 = jax.nn.relu(_gcn(x, src, dst, w0, Wd0, bd0, N_NODES))
    x1, s1, d1, w1, perm1 = _topk_pool(h0, src, dst, w0, p1, K1, N_NODES)
    h1 = jax.nn.relu(_gcn(x1, s1, d1, w1, Wd1, bd1, K1))
    x2, s2, d2, w2, perm2 = _topk_pool(h1, s1, d1, w1, p2, K2, K1)
    h2 = jax.nn.relu(_gcn(x2, s2, d2, w2, Wd2, bd2, K2))
    up1 = jnp.zeros((K1, HID), dtype=jnp.float32).at[perm2].set(h2)
    u1 = h1 + up1
    u1 = jax.nn.relu(_gcn(u1, s1, d1, w1, Wu0, bu0, K1))
    up0 = jnp.zeros((N_NODES, HID), dtype=jnp.float32).at[perm1].set(u1)
    u0 = h0 + up0
    u0 = _gcn(u0, src, dst, w0, Wu1, bu1, N_NODES)
    sums = jax.ops.segment_sum(u0, batch.astype(jnp.int32), num_segments=N_GRAPHS)
    cnt = jax.ops.segment_sum(jnp.ones((N_NODES,), dtype=jnp.float32), batch.astype(jnp.int32), num_segments=N_GRAPHS)
    g = sums / jnp.maximum(cnt, 1.0)[:, None]
    return _head(g, Wm1, bm1, Wm2, bm2)


# speedup vs baseline: 7.4309x; 7.4309x over previous
import functools

import jax
import jax.numpy as jnp
from jax import lax
from jax.experimental import pallas as pl
from jax.experimental.pallas import tpu as pltpu
from jax.experimental.pallas import tpu_sc as plsc

N_NODES = 10000
N_EDGES = 160000
D_IN = 256
HID = 512
NHID = 1024
NOUT = 128
N_GRAPHS = 64
K1 = 5000
K2 = 2500

NP = 10240
NPR = NP // 128
RB = 1280
NSC = 2
NTL = 16
ROWS_PER_TILE = NP // NTL
E_PER_TILE_SEG = N_EDGES // NTL
E_PER_TILE_DEG = N_EDGES // (NSC * NTL)
EB = 128


def _mesh():
    return plsc.VectorSubcoreMesh(
        core_axis_name="c", subcore_axis_name="s",
        num_cores=NSC, num_subcores=NTL)


def _deg_body(mask_hbm, src_hbm, dst_hbm, out_hbm,
              idxs, idxd, vals, zbuf, mask_sh, deg_sh, sem1, sem2):
    c = lax.axis_index("c")
    s = lax.axis_index("s")

    def zset(i, _):
        zbuf[pl.ds(i * 16, 16)] = jnp.zeros((16,), jnp.float32)
        return 0
    lax.fori_loop(0, ROWS_PER_TILE // 16, zset, 0)
    pltpu.sync_copy(zbuf, deg_sh.at[pl.ds(s * ROWS_PER_TILE, ROWS_PER_TILE)])
    @pl.when(s == 0)
    def _():
        pltpu.sync_copy(mask_hbm, mask_sh)
    plsc.subcore_barrier()

    base = (c * NTL + s) * E_PER_TILE_DEG

    def chunk(off, n):
        pltpu.sync_copy(src_hbm.at[pl.ds(off, n)], idxs.at[pl.ds(0, n)])
        pltpu.sync_copy(dst_hbm.at[pl.ds(off, n)], idxd.at[pl.ds(0, n)])
        pltpu.async_copy(mask_sh.at[idxs.at[pl.ds(0, n)]],
                         vals.at[pl.ds(0, n)], sem1).wait()
        pltpu.async_copy(vals.at[pl.ds(0, n)],
                         deg_sh.at[idxd.at[pl.ds(0, n)]], sem2, add=True).wait()

    def loop(i, _):
        chunk(base + i * EB, EB)
        return 0
    lax.fori_loop(0, E_PER_TILE_DEG // EB, loop, 0)
    chunk(base + (E_PER_TILE_DEG // EB) * EB, E_PER_TILE_DEG % EB)

    plsc.subcore_barrier()
    pltpu.sync_copy(deg_sh.at[pl.ds(s * ROWS_PER_TILE, ROWS_PER_TILE)],
                    out_hbm.at[c, pl.ds(s * ROWS_PER_TILE, ROWS_PER_TILE)])


def _deg(mask, src, dst):
    f = pl.kernel(
        _deg_body,
        out_type=jax.ShapeDtypeStruct((NSC, NP), jnp.float32),
        mesh=_mesh(),
        scratch_types=[
            pltpu.VMEM((EB,), jnp.int32),
            pltpu.VMEM((EB,), jnp.int32),
            pltpu.VMEM((EB,), jnp.float32),
            pltpu.VMEM((ROWS_PER_TILE,), jnp.float32),
            pltpu.VMEM_SHARED((NP,), jnp.float32),
            pltpu.VMEM_SHARED((NP,), jnp.float32),
            pltpu.SemaphoreType.DMA,
            pltpu.SemaphoreType.DMA,
        ],
    )
    return f(mask, src, dst)


def _seg_body(hp0, hp1, hp2, hp3, src_hbm, dst_hbm, o0, o1, o2, o3,
              idxs, idxd, rows, zb, acc, sem1, sem2):
    c = lax.axis_index("c")
    s = lax.axis_index("s")

    def zset(i, _):
        r = i // 8
        j = i % 8
        zb[r, pl.ds(j * 16, 16)] = jnp.zeros((16,), jnp.float32)
        return 0
    lax.fori_loop(0, 64 * 8, zset, 0)

    def one_block(hp_in, o_out):
        def zcp(j, _):
            pltpu.sync_copy(zb, acc.at[pl.ds(s * ROWS_PER_TILE + j * 64, 64), :])
            return 0
        lax.fori_loop(0, ROWS_PER_TILE // 64, zcp, 0)
        plsc.subcore_barrier()

        base = s * E_PER_TILE_SEG

        def chunk(off, n):
            pltpu.sync_copy(src_hbm.at[pl.ds(off, n)], idxs.at[pl.ds(0, n)])
            pltpu.sync_copy(dst_hbm.at[pl.ds(off, n)], idxd.at[pl.ds(0, n)])
            pltpu.async_copy(hp_in.at[idxs.at[pl.ds(0, n)]],
                             rows.at[pl.ds(0, n), :], sem1).wait()
            pltpu.async_copy(rows.at[pl.ds(0, n), :],
                             acc.at[idxd.at[pl.ds(0, n)]], sem2, add=True).wait()

        def loop(i, _):
            chunk(base + i * EB, EB)
            return 0
        lax.fori_loop(0, E_PER_TILE_SEG // EB, loop, 0)
        chunk(base + (E_PER_TILE_SEG // EB) * EB, E_PER_TILE_SEG % EB)

        plsc.subcore_barrier()
        pltpu.sync_copy(acc.at[pl.ds(s * ROWS_PER_TILE, ROWS_PER_TILE), :],
                        o_out.at[pl.ds(s * ROWS_PER_TILE, ROWS_PER_TILE), :])
        plsc.subcore_barrier()

    @pl.when(c == 0)
    def _():
        one_block(hp0, o0)
        one_block(hp1, o1)

    @pl.when(c == 1)
    def _():
        one_block(hp2, o2)
        one_block(hp3, o3)


def _seg(hps, src, dst):
    f = pl.kernel(
        _seg_body,
        out_type=tuple(jax.ShapeDtypeStruct((NP, 128), jnp.float32)
                       for _ in range(4)),
        mesh=_mesh(),
        scratch_types=[
            pltpu.VMEM((EB,), jnp.int32),
            pltpu.VMEM((EB,), jnp.int32),
            pltpu.VMEM((EB, 128), jnp.float32),
            pltpu.VMEM((64, 128), jnp.float32),
            pltpu.VMEM_SHARED((NP, 128), jnp.float32),
            pltpu.SemaphoreType.DMA,
            pltpu.SemaphoreType.DMA,
        ],
    )
    return f(hps[0], hps[1], hps[2], hps[3], src, dst)


def _mm_body1(xa_ref, sa_ref, w_ref, degt_ref, o0, o1, o2, o3):
    x = xa_ref[...] * sa_ref[...]
    h = jnp.dot(x, w_ref[...], preferred_element_type=jnp.float32)
    dinv = lax.rsqrt(degt_ref[...].sum(axis=1, keepdims=True) + 2.0)
    h = h * dinv
    o0[...] = h[:, 0:128]
    o1[...] = h[:, 128:256]
    o2[...] = h[:, 256:384]
    o3[...] = h[:, 384:512]


def _mm_body2(xa_ref, sa_ref, xb_ref, sb_ref, w_ref, degt_ref, o0, o1, o2, o3):
    x = xa_ref[...] * sa_ref[...] + xb_ref[...] * sb_ref[...]
    h = jnp.dot(x, w_ref[...], preferred_element_type=jnp.float32)
    dinv = lax.rsqrt(degt_ref[...].sum(axis=1, keepdims=True) + 2.0)
    h = h * dinv
    o0[...] = h[:, 0:128]
    o1[...] = h[:, 128:256]
    o2[...] = h[:, 256:384]
    o3[...] = h[:, 384:512]


def _mm_scale(xa, sa, W, degt, xb=None, sb=None):
    din = xa.shape[1]
    grid = (NP // RB,)
    row_spec = pl.BlockSpec((RB, din), lambda r: (r, 0))
    col_spec = pl.BlockSpec((RB, 1), lambda r: (r, 0))
    w_spec = pl.BlockSpec((din, HID), lambda r: (0, 0))
    deg_spec = pl.BlockSpec((RB, 2), lambda r: (r, 0))
    out_specs = tuple(pl.BlockSpec((RB, 128), lambda r: (r, 0)) for _ in range(4))
    out_shape = tuple(jax.ShapeDtypeStruct((NP, 128), jnp.float32) for _ in range(4))
    if xb is None:
        return pl.pallas_call(
            _mm_body1, grid=grid,
            in_specs=[row_spec, col_spec, w_spec, deg_spec],
            out_specs=out_specs, out_shape=out_shape,
        )(xa, sa, W, degt)
    return pl.pallas_call(
        _mm_body2, grid=grid,
        in_specs=[row_spec, col_spec, row_spec, col_spec, w_spec, deg_spec],
        out_specs=out_specs, out_shape=out_shape,
    )(xa, sa, xb, sb, W, degt)


def _comb_body(s0, s1, s2, s3, h0, h1, h2, h3, degt_ref, b_ref, o_ref, *, relu):
    dinv = lax.rsqrt(degt_ref[...].sum(axis=1, keepdims=True) + 2.0)
    parts = []
    for s_ref, h_ref in ((s0, h0), (s1, h1), (s2, h2), (s3, h3)):
        parts.append(dinv * (s_ref[...] + 2.0 * h_ref[...]))
    v = jnp.concatenate(parts, axis=1) + b_ref[...]
    if relu:
        v = jnp.maximum(v, 0.0)
    row = (pl.program_id(0) * RB
           + lax.broadcasted_iota(jnp.int32, (RB, 1), 0))
    o_ref[...] = jnp.where(row < N_NODES, v, 0.0)


def _combine(segs, hps, degt, b, relu):
    grid = (NP // RB,)
    blk = pl.BlockSpec((RB, 128), lambda r: (r, 0))
    deg_spec = pl.BlockSpec((RB, 2), lambda r: (r, 0))
    b_spec = pl.BlockSpec((1, HID), lambda r: (0, 0))
    return pl.pallas_call(
        functools.partial(_comb_body, relu=relu), grid=grid,
        in_specs=[blk] * 8 + [deg_spec, b_spec],
        out_specs=pl.BlockSpec((RB, HID), lambda r: (r, 0)),
        out_shape=jax.ShapeDtypeStruct((NP, HID), jnp.float32),
    )(*segs, *hps, degt, b.reshape(1, HID))


def _sortkey(score):
    u = lax.bitcast_convert_type(score, jnp.uint32)
    i = lax.bitcast_convert_type(score, jnp.int32)
    return jnp.where(i >= 0, u + jnp.uint32(0x80000000), ~u)


def _radix_max_ge(valid, keys, r, nbits):
    def body(i, t):
        b = nbits - 1 - i
        cand = t | (jnp.uint32(1) << jnp.uint32(b))
        cnt = jnp.sum(jnp.where(valid & (keys >= cand), 1.0, 0.0))
        return jnp.where(cnt >= r, cand, t)
    return lax.fori_loop(0, nbits, body, jnp.uint32(0))


def _pool_body(h3_ref, p_ref, mprev_ref, ks_ref, m_ref, sm_ref, key_ref, *, k):
    p = p_ref[...]
    norm = jnp.sqrt(jnp.sum(p * p))
    xp = jnp.sum(h3_ref[...] * p[None, :, :].reshape(1, 1, HID), axis=2)
    score = jnp.tanh(xp / norm)
    elig = mprev_ref[...] > 0.0
    ukey = jnp.where(elig, _sortkey(score), jnp.uint32(0))

    kf = jnp.float32(k)
    K = _radix_max_ge(elig, ukey, kf, 32)
    g = jnp.sum(jnp.where(ukey > K, 1.0, 0.0))
    r = kf - g
    tie = ukey == K

    ks = ks_ref[...]
    S = _radix_max_ge(tie, ks, r, 32)
    S = jnp.where(r > 0, S, jnp.uint32(0xFFFFFFFF))
    g2 = jnp.sum(jnp.where(tie & (ks > S), 1.0, 0.0))
    r2 = r - g2
    tie2 = tie & (ks == S)

    idx = (lax.broadcasted_iota(jnp.int32, (NPR, 128), 0) * 128
           + lax.broadcasted_iota(jnp.int32, (NPR, 128), 1))
    idxp = jnp.uint32(16383) - idx.astype(jnp.uint32)
    P = _radix_max_ge(tie2, idxp, r2, 14)
    P = jnp.where(r2 > 0, P, jnp.uint32(16384))

    sel = (ukey > K) | (tie & (ks > S)) | (tie2 & (idxp >= P))
    m = jnp.where(sel, 1.0, 0.0)
    m_ref[...] = m
    sm_ref[...] = score * m
    key_ref[...] = ukey


def _pool(h, p, mprev, ks, k):
    h3 = h.reshape(NPR, 128, HID)
    out = pl.pallas_call(
        functools.partial(_pool_body, k=k),
        out_shape=(
            jax.ShapeDtypeStruct((NPR, 128), jnp.float32),
            jax.ShapeDtypeStruct((NPR, 128), jnp.float32),
            jax.ShapeDtypeStruct((NPR, 128), jnp.uint32),
        ),
    )(h3, p.reshape(1, HID), mprev, ks)
    return out


def _head_body(batch_ref, u_ref, w1_ref, b1_ref, w2_ref, b2_ref, o_ref):
    b_row = batch_ref[...]
    gid = lax.broadcasted_iota(jnp.int32, (N_GRAPHS, NP), 0)
    Bm = (b_row == gid).astype(jnp.float32)
    sums = jnp.dot(Bm, u_ref[...], preferred_element_type=jnp.float32)
    cnt = jnp.sum(Bm, axis=1, keepdims=True)
    g = sums / jnp.maximum(cnt, 1.0)
    hdn = jnp.maximum(
        jnp.dot(g, w1_ref[...], preferred_element_type=jnp.float32)
        + b1_ref[...], 0.0)
    o_ref[...] = jnp.dot(hdn, w2_ref[...],
                         preferred_element_type=jnp.float32) + b2_ref[...]


def _head(batch_p, u0, Wm1, bm1, Wm2, bm2):
    return pl.pallas_call(
        _head_body,
        out_shape=jax.ShapeDtypeStruct((N_GRAPHS, NOUT), jnp.float32),
    )(batch_p, u0, Wm1, bm1.reshape(1, NHID), Wm2, bm2.reshape(1, NOUT))


def _pad_rows(a):
    return jnp.pad(a, ((0, NP - a.shape[0]), (0, 0)))


def kernel(x, edge_index, batch, Wd0, bd0, p1, Wd1, bd1, p2, Wd2, bd2,
           Wu0, bu0, Wu1, bu1, Wm1, bm1, Wm2, bm2):
    src = edge_index[0].astype(jnp.int32)
    dst = edge_index[1].astype(jnp.int32)
    xp = _pad_rows(x)
    ones_col = jnp.ones((NP, 1), jnp.float32)
    real = (jnp.arange(NP, dtype=jnp.int32) < N_NODES)
    ones_real = real.astype(jnp.float32)
    elig0 = ones_real.reshape(NPR, 128)
    zeros_key = jnp.zeros((NPR, 128), jnp.uint32)

    deg0t = _deg(ones_real, src, dst).T
    hp0 = _mm_scale(xp, ones_col, Wd0, deg0t)
    seg0 = _seg(hp0, src, dst)
    h0 = _combine(seg0, hp0, deg0t, bd0, relu=True)

    m1, sm1, key1 = _pool(h0, p1, elig0, zeros_key, K1)
    m1f = m1.reshape(NP)
    deg1t = _deg(m1f, src, dst).T
    hp1 = _mm_scale(h0, sm1.reshape(NP, 1), Wd1, deg1t)
    seg1 = _seg(hp1, src, dst)
    h1 = _combine(seg1, hp1, deg1t, bd1, relu=True)

    m2, sm2, key2 = _pool(h1, p2, m1, key1, K2)
    m2f = m2.reshape(NP)
    deg2t = _deg(m2f, src, dst).T
    hp2 = _mm_scale(h1, sm2.reshape(NP, 1), Wd2, deg2t)
    seg2 = _seg(hp2, src, dst)
    h2 = _combine(seg2, hp2, deg2t, bd2, relu=True)

    hpu1 = _mm_scale(h1, m1.reshape(NP, 1), Wu0, deg1t,
                     xb=h2, sb=m2.reshape(NP, 1))
    segu1 = _seg(hpu1, src, dst)
    u1c = _combine(segu1, hpu1, deg1t, bu0, relu=True)

    hpu0 = _mm_scale(h0, ones_col, Wu1, deg0t,
                     xb=u1c, sb=m1.reshape(NP, 1))
    segu0 = _seg(hpu0, src, dst)
    u0 = _combine(segu0, hpu0, deg0t, bu1, relu=False)

    batch_p = jnp.pad(batch.astype(jnp.int32), (0, NP - N_NODES),
                      constant_values=N_GRAPHS).reshape(1, NP)
    return _head(batch_p, u0, Wm1, bm1, Wm2, bm2)


# speedup vs baseline: 11.9343x; 1.6060x over previous
import functools

import jax
import jax.numpy as jnp
from jax import lax
from jax.experimental import pallas as pl
from jax.experimental.pallas import tpu as pltpu
from jax.experimental.pallas import tpu_sc as plsc

N_NODES = 10000
N_EDGES = 160000
D_IN = 256
HID = 512
NHID = 1024
NOUT = 128
N_GRAPHS = 64
K1 = 5000
K2 = 2500

NP = 10240
NPR = NP // 128
RB = 1280
NSC = 2
NTL = 16
ROWS_PER_TILE = NP // NTL
E_PER_TILE_SEG = N_EDGES // NTL
E_PER_TILE_DEG = N_EDGES // (NSC * NTL)
EB = 128


def _mesh():
    return plsc.VectorSubcoreMesh(
        core_axis_name="c", subcore_axis_name="s",
        num_cores=NSC, num_subcores=NTL)


def _deg_body(mask_hbm, src_hbm, dst_hbm, out_hbm,
              idxs, idxd, vals, zbuf, mask_sh, deg_sh, sem1, sem2):
    c = lax.axis_index("c")
    s = lax.axis_index("s")

    def zset(i, _):
        zbuf[pl.ds(i * 16, 16)] = jnp.zeros((16,), jnp.float32)
        return 0
    lax.fori_loop(0, ROWS_PER_TILE // 16, zset, 0)
    pltpu.sync_copy(zbuf, deg_sh.at[pl.ds(s * ROWS_PER_TILE, ROWS_PER_TILE)])
    @pl.when(s == 0)
    def _():
        pltpu.sync_copy(mask_hbm, mask_sh)
    plsc.subcore_barrier()

    base = (c * NTL + s) * E_PER_TILE_DEG

    def chunk(off, n):
        pltpu.sync_copy(src_hbm.at[pl.ds(off, n)], idxs.at[pl.ds(0, n)])
        pltpu.sync_copy(dst_hbm.at[pl.ds(off, n)], idxd.at[pl.ds(0, n)])
        pltpu.async_copy(mask_sh.at[idxs.at[pl.ds(0, n)]],
                         vals.at[pl.ds(0, n)], sem1).wait()
        pltpu.async_copy(vals.at[pl.ds(0, n)],
                         deg_sh.at[idxd.at[pl.ds(0, n)]], sem2, add=True).wait()

    def loop(i, _):
        chunk(base + i * EB, EB)
        return 0
    lax.fori_loop(0, E_PER_TILE_DEG // EB, loop, 0)
    chunk(base + (E_PER_TILE_DEG // EB) * EB, E_PER_TILE_DEG % EB)

    plsc.subcore_barrier()
    pltpu.sync_copy(deg_sh.at[pl.ds(s * ROWS_PER_TILE, ROWS_PER_TILE)],
                    out_hbm.at[c, pl.ds(s * ROWS_PER_TILE, ROWS_PER_TILE)])


def _deg(mask, src, dst):
    f = pl.kernel(
        _deg_body,
        out_type=jax.ShapeDtypeStruct((NSC, NP), jnp.float32),
        mesh=_mesh(),
        scratch_types=[
            pltpu.VMEM((EB,), jnp.int32),
            pltpu.VMEM((EB,), jnp.int32),
            pltpu.VMEM((EB,), jnp.float32),
            pltpu.VMEM((ROWS_PER_TILE,), jnp.float32),
            pltpu.VMEM_SHARED((NP,), jnp.float32),
            pltpu.VMEM_SHARED((NP,), jnp.float32),
            pltpu.SemaphoreType.DMA,
            pltpu.SemaphoreType.DMA,
        ],
    )
    return f(mask, src, dst)


E_T = 9984
NCH = 78
E_ROWS = N_EDGES // 128


def _seg_body(hp0, hp1, hp2, hp3, src_hbm, dst_hbm, o0, o1, o2, o3,
              sidx, sidx_x, didx, rows, zb, acc,
              gs0, gs1, ss0, ss1, ds0, ds1):
    c = lax.axis_index("c")
    s = lax.axis_index("s")
    gsem = (gs0, gs1)
    ssem = (ss0, ss1)
    dsem = (ds0, ds1)

    def zset(i, _):
        zb[i // 8, pl.ds((i % 8) * 16, 16)] = jnp.zeros((16,), jnp.float32)
        return 0
    lax.fori_loop(0, 16 * 8, zset, 0)

    pltpu.sync_copy(src_hbm.at[pl.ds(s * E_T, E_T)], sidx)

    @pl.when(s == 0)
    def _():
        pltpu.sync_copy(src_hbm.at[pl.ds(NTL * E_T, 256)], sidx_x)

    def one_block(hp_in, o_out):
        def zcp(j, _):
            pltpu.sync_copy(zb, acc.at[pl.ds(s * ROWS_PER_TILE + j * 16, 16), :])
            return 0
        lax.fori_loop(0, ROWS_PER_TILE // 16, zcp, 0)
        plsc.subcore_barrier()

        g_d = [None, None]
        s_d = [None, None]
        d_d = [None, None]
        for i in range(NCH):
            b = i & 1
            if s_d[b] is not None:
                s_d[b].wait()
            d_d[b] = pltpu.async_copy(
                dst_hbm.at[pl.ds(s * E_T + i * 128, 128)], didx.at[b], dsem[b])
            g_d[b] = pltpu.async_copy(
                hp_in.at[sidx.at[pl.ds(i * 128, 128)]], rows.at[b], gsem[b])
            g_d[b].wait()
            d_d[b].wait()
            s_d[b] = pltpu.async_copy(
                rows.at[b], acc.at[didx.at[b]], ssem[b], add=True)
        s_d[0].wait()
        s_d[1].wait()

        @pl.when(s == 0)
        def _():
            for j in range(2):
                pltpu.async_copy(dst_hbm.at[pl.ds(NTL * E_T + j * 128, 128)],
                                 didx.at[0], dsem[0]).wait()
                pltpu.async_copy(hp_in.at[sidx_x.at[pl.ds(j * 128, 128)]],
                                 rows.at[0], gsem[0]).wait()
                pltpu.async_copy(rows.at[0], acc.at[didx.at[0]],
                                 ssem[0], add=True).wait()

        plsc.subcore_barrier()
        pltpu.sync_copy(acc.at[pl.ds(s * ROWS_PER_TILE, ROWS_PER_TILE), :],
                        o_out.at[pl.ds(s * ROWS_PER_TILE, ROWS_PER_TILE), :])
        plsc.subcore_barrier()

    @pl.when(c == 0)
    def _():
        one_block(hp0, o0)
        one_block(hp1, o1)

    @pl.when(c == 1)
    def _():
        one_block(hp2, o2)
        one_block(hp3, o3)


def _seg(hps, src, dst):
    f = pl.kernel(
        _seg_body,
        out_type=tuple(jax.ShapeDtypeStruct((NP, 128), jnp.float32)
                       for _ in range(4)),
        mesh=_mesh(),
        scratch_types=[
            pltpu.VMEM((E_T,), jnp.int32),
            pltpu.VMEM((256,), jnp.int32),
            pltpu.VMEM((2, 128), jnp.int32),
            pltpu.VMEM((2, 128, 128), jnp.float32),
            pltpu.VMEM((16, 128), jnp.float32),
            pltpu.VMEM_SHARED((NP, 128), jnp.float32),
            pltpu.SemaphoreType.DMA,
            pltpu.SemaphoreType.DMA,
            pltpu.SemaphoreType.DMA,
            pltpu.SemaphoreType.DMA,
            pltpu.SemaphoreType.DMA,
            pltpu.SemaphoreType.DMA,
        ],
    )
    return f(hps[0], hps[1], hps[2], hps[3], src, dst)


def _mm_body1(xa_ref, sa_ref, w_ref, degt_ref, o0, o1, o2, o3):
    x = xa_ref[...] * sa_ref[...]
    h = jnp.dot(x, w_ref[...], preferred_element_type=jnp.float32)
    dinv = lax.rsqrt(degt_ref[...].sum(axis=1, keepdims=True) + 2.0)
    h = h * dinv
    o0[...] = h[:, 0:128]
    o1[...] = h[:, 128:256]
    o2[...] = h[:, 256:384]
    o3[...] = h[:, 384:512]


def _mm_body2(xa_ref, sa_ref, xb_ref, sb_ref, w_ref, degt_ref, o0, o1, o2, o3):
    x = xa_ref[...] * sa_ref[...] + xb_ref[...] * sb_ref[...]
    h = jnp.dot(x, w_ref[...], preferred_element_type=jnp.float32)
    dinv = lax.rsqrt(degt_ref[...].sum(axis=1, keepdims=True) + 2.0)
    h = h * dinv
    o0[...] = h[:, 0:128]
    o1[...] = h[:, 128:256]
    o2[...] = h[:, 256:384]
    o3[...] = h[:, 384:512]


def _mm_scale(xa, sa, W, degt, xb=None, sb=None):
    din = xa.shape[1]
    grid = (NP // RB,)
    row_spec = pl.BlockSpec((RB, din), lambda r: (r, 0))
    col_spec = pl.BlockSpec((RB, 1), lambda r: (r, 0))
    w_spec = pl.BlockSpec((din, HID), lambda r: (0, 0))
    deg_spec = pl.BlockSpec((RB, 2), lambda r: (r, 0))
    out_specs = tuple(pl.BlockSpec((RB, 128), lambda r: (r, 0)) for _ in range(4))
    out_shape = tuple(jax.ShapeDtypeStruct((NP, 128), jnp.float32) for _ in range(4))
    if xb is None:
        return pl.pallas_call(
            _mm_body1, grid=grid,
            in_specs=[row_spec, col_spec, w_spec, deg_spec],
            out_specs=out_specs, out_shape=out_shape,
        )(xa, sa, W, degt)
    return pl.pallas_call(
        _mm_body2, grid=grid,
        in_specs=[row_spec, col_spec, row_spec, col_spec, w_spec, deg_spec],
        out_specs=out_specs, out_shape=out_shape,
    )(xa, sa, xb, sb, W, degt)


def _comb_body(s0, s1, s2, s3, h0, h1, h2, h3, degt_ref, b_ref, o_ref, *, relu):
    dinv = lax.rsqrt(degt_ref[...].sum(axis=1, keepdims=True) + 2.0)
    parts = []
    for s_ref, h_ref in ((s0, h0), (s1, h1), (s2, h2), (s3, h3)):
        parts.append(dinv * (s_ref[...] + 2.0 * h_ref[...]))
    v = jnp.concatenate(parts, axis=1) + b_ref[...]
    if relu:
        v = jnp.maximum(v, 0.0)
    row = (pl.program_id(0) * RB
           + lax.broadcasted_iota(jnp.int32, (RB, 1), 0))
    o_ref[...] = jnp.where(row < N_NODES, v, 0.0)


def _combine(segs, hps, degt, b, relu):
    grid = (NP // RB,)
    blk = pl.BlockSpec((RB, 128), lambda r: (r, 0))
    deg_spec = pl.BlockSpec((RB, 2), lambda r: (r, 0))
    b_spec = pl.BlockSpec((1, HID), lambda r: (0, 0))
    return pl.pallas_call(
        functools.partial(_comb_body, relu=relu), grid=grid,
        in_specs=[blk] * 8 + [deg_spec, b_spec],
        out_specs=pl.BlockSpec((RB, HID), lambda r: (r, 0)),
        out_shape=jax.ShapeDtypeStruct((NP, HID), jnp.float32),
    )(*segs, *hps, degt, b.reshape(1, HID))


def _sortkey(score):
    u = lax.bitcast_convert_type(score, jnp.uint32)
    i = lax.bitcast_convert_type(score, jnp.int32)
    return jnp.where(i >= 0, u + jnp.uint32(0x80000000), ~u)


def _radix_max_ge(valid, keys, r, nbits):
    def body(i, t):
        b = nbits - 1 - i
        cand = t | (jnp.uint32(1) << jnp.uint32(b))
        cnt = jnp.sum(jnp.where(valid & (keys >= cand), 1.0, 0.0))
        return jnp.where(cnt >= r, cand, t)
    return lax.fori_loop(0, nbits, body, jnp.uint32(0))


def _pool_body(h3_ref, p_ref, mprev_ref, ks_ref, m_ref, sm_ref, key_ref, *, k):
    p = p_ref[...]
    norm = jnp.sqrt(jnp.sum(p * p))
    xp = jnp.sum(h3_ref[...] * p[None, :, :].reshape(1, 1, HID), axis=2)
    score = jnp.tanh(xp / norm)
    elig = mprev_ref[...] > 0.0
    ukey = jnp.where(elig, _sortkey(score), jnp.uint32(0))

    kf = jnp.float32(k)
    K = _radix_max_ge(elig, ukey, kf, 32)
    g = jnp.sum(jnp.where(ukey > K, 1.0, 0.0))
    r = kf - g
    tie = ukey == K

    ks = ks_ref[...]
    S = _radix_max_ge(tie, ks, r, 32)
    S = jnp.where(r > 0, S, jnp.uint32(0xFFFFFFFF))
    g2 = jnp.sum(jnp.where(tie & (ks > S), 1.0, 0.0))
    r2 = r - g2
    tie2 = tie & (ks == S)

    idx = (lax.broadcasted_iota(jnp.int32, (NPR, 128), 0) * 128
           + lax.broadcasted_iota(jnp.int32, (NPR, 128), 1))
    idxp = jnp.uint32(16383) - idx.astype(jnp.uint32)
    P = _radix_max_ge(tie2, idxp, r2, 14)
    P = jnp.where(r2 > 0, P, jnp.uint32(16384))

    sel = (ukey > K) | (tie & (ks > S)) | (tie2 & (idxp >= P))
    m = jnp.where(sel, 1.0, 0.0)
    m_ref[...] = m
    sm_ref[...] = score * m
    key_ref[...] = ukey


def _pool(h, p, mprev, ks, k):
    h3 = h.reshape(NPR, 128, HID)
    out = pl.pallas_call(
        functools.partial(_pool_body, k=k),
        out_shape=(
            jax.ShapeDtypeStruct((NPR, 128), jnp.float32),
            jax.ShapeDtypeStruct((NPR, 128), jnp.float32),
            jax.ShapeDtypeStruct((NPR, 128), jnp.uint32),
        ),
    )(h3, p.reshape(1, HID), mprev, ks)
    return out


def _head_body(batch_ref, u_ref, w1_ref, b1_ref, w2_ref, b2_ref, o_ref):
    b_row = batch_ref[...]
    gid = lax.broadcasted_iota(jnp.int32, (N_GRAPHS, NP), 0)
    Bm = (b_row == gid).astype(jnp.float32)
    sums = jnp.dot(Bm, u_ref[...], preferred_element_type=jnp.float32)
    cnt = jnp.sum(Bm, axis=1, keepdims=True)
    g = sums / jnp.maximum(cnt, 1.0)
    hdn = jnp.maximum(
        jnp.dot(g, w1_ref[...], preferred_element_type=jnp.float32)
        + b1_ref[...], 0.0)
    o_ref[...] = jnp.dot(hdn, w2_ref[...],
                         preferred_element_type=jnp.float32) + b2_ref[...]


def _head(batch_p, u0, Wm1, bm1, Wm2, bm2):
    return pl.pallas_call(
        _head_body,
        out_shape=jax.ShapeDtypeStruct((N_GRAPHS, NOUT), jnp.float32),
    )(batch_p, u0, Wm1, bm1.reshape(1, NHID), Wm2, bm2.reshape(1, NOUT))


def _pad_rows(a):
    return jnp.pad(a, ((0, NP - a.shape[0]), (0, 0)))


def kernel(x, edge_index, batch, Wd0, bd0, p1, Wd1, bd1, p2, Wd2, bd2,
           Wu0, bu0, Wu1, bu1, Wm1, bm1, Wm2, bm2):
    src = edge_index[0].astype(jnp.int32)
    dst = edge_index[1].astype(jnp.int32)
    xp = _pad_rows(x)
    ones_col = jnp.ones((NP, 1), jnp.float32)
    real = (jnp.arange(NP, dtype=jnp.int32) < N_NODES)
    ones_real = real.astype(jnp.float32)
    elig0 = ones_real.reshape(NPR, 128)
    zeros_key = jnp.zeros((NPR, 128), jnp.uint32)

    deg0t = _deg(ones_real, src, dst).T
    hp0 = _mm_scale(xp, ones_col, Wd0, deg0t)
    seg0 = _seg(hp0, src, dst)
    h0 = _combine(seg0, hp0, deg0t, bd0, relu=True)

    m1, sm1, key1 = _pool(h0, p1, elig0, zeros_key, K1)
    m1f = m1.reshape(NP)
    deg1t = _deg(m1f, src, dst).T
    hp1 = _mm_scale(h0, sm1.reshape(NP, 1), Wd1, deg1t)
    seg1 = _seg(hp1, src, dst)
    h1 = _combine(seg1, hp1, deg1t, bd1, relu=True)

    m2, sm2, key2 = _pool(h1, p2, m1, key1, K2)
    m2f = m2.reshape(NP)
    deg2t = _deg(m2f, src, dst).T
    hp2 = _mm_scale(h1, sm2.reshape(NP, 1), Wd2, deg2t)
    seg2 = _seg(hp2, src, dst)
    h2 = _combine(seg2, hp2, deg2t, bd2, relu=True)

    hpu1 = _mm_scale(h1, m1.reshape(NP, 1), Wu0, deg1t,
                     xb=h2, sb=m2.reshape(NP, 1))
    segu1 = _seg(hpu1, src, dst)
    u1c = _combine(segu1, hpu1, deg1t, bu0, relu=True)

    hpu0 = _mm_scale(h0, ones_col, Wu1, deg0t,
                     xb=u1c, sb=m1.reshape(NP, 1))
    segu0 = _seg(hpu0, src, dst)
    u0 = _combine(segu0, hpu0, deg0t, bu1, relu=False)

    batch_p = jnp.pad(batch.astype(jnp.int32), (0, NP - N_NODES),
                      constant_values=N_GRAPHS).reshape(1, NP)
    return _head(batch_p, u0, Wm1, bm1, Wm2, bm2)


# speedup vs baseline: 14.7427x; 1.2353x over previous
import functools

import jax
import jax.numpy as jnp
from jax import lax
from jax.experimental import pallas as pl
from jax.experimental.pallas import tpu as pltpu
from jax.experimental.pallas import tpu_sc as plsc

N_NODES = 10000
N_EDGES = 160000
D_IN = 256
HID = 512
NHID = 1024
NOUT = 128
N_GRAPHS = 64
K1 = 5000
K2 = 2500

NP = 10240
NPR = NP // 128
RB = 1280
NSC = 2
NTL = 16
ROWS_PER_TILE = NP // NTL
E_PER_TILE_SEG = N_EDGES // NTL
E_PER_TILE_DEG = N_EDGES // (NSC * NTL)
EB = 128


def _mesh():
    return plsc.VectorSubcoreMesh(
        core_axis_name="c", subcore_axis_name="s",
        num_cores=NSC, num_subcores=NTL)


E_T_DEG = 4992
NCH_DEG = 39


def _deg_body(mask_hbm, src_hbm, dst_hbm, out_hbm,
              sidx, didx, vals, zbuf, mask_sh, deg_sh,
              gs0, gs1, ss0, ss1, ds0, ds1):
    c = lax.axis_index("c")
    s = lax.axis_index("s")
    gsem = (gs0, gs1)
    ssem = (ss0, ss1)
    dsem = (ds0, ds1)

    def zset(i, _):
        zbuf[pl.ds(i * 16, 16)] = jnp.zeros((16,), jnp.float32)
        return 0
    lax.fori_loop(0, ROWS_PER_TILE // 16, zset, 0)
    pltpu.sync_copy(zbuf, deg_sh.at[pl.ds(s * ROWS_PER_TILE, ROWS_PER_TILE)])
    @pl.when(s == 0)
    def _():
        pltpu.sync_copy(mask_hbm, mask_sh)

    tile = c * NTL + s
    base = tile * E_T_DEG
    pltpu.sync_copy(src_hbm.at[pl.ds(base, E_T_DEG)], sidx)
    plsc.subcore_barrier()

    g_d = [None, None]
    s_d = [None, None]
    d_d = [None, None]
    for i in range(NCH_DEG):
        b = i & 1
        if s_d[b] is not None:
            s_d[b].wait()
        d_d[b] = pltpu.async_copy(
            dst_hbm.at[pl.ds(base + i * 128, 128)], didx.at[b], dsem[b])
        g_d[b] = pltpu.async_copy(
            mask_sh.at[sidx.at[pl.ds(i * 128, 128)]], vals.at[b], gsem[b])
        g_d[b].wait()
        d_d[b].wait()
        s_d[b] = pltpu.async_copy(
            vals.at[b], deg_sh.at[didx.at[b]], ssem[b], add=True)
    s_d[0].wait()
    s_d[1].wait()

    @pl.when(s == 0)
    def _():
        off = NSC * NTL * E_T_DEG + c * 128
        pltpu.sync_copy(src_hbm.at[pl.ds(off, 128)], sidx.at[pl.ds(0, 128)])
        pltpu.async_copy(dst_hbm.at[pl.ds(off, 128)], didx.at[0], dsem[0]).wait()
        pltpu.async_copy(mask_sh.at[sidx.at[pl.ds(0, 128)]],
                         vals.at[0], gsem[0]).wait()
        pltpu.async_copy(vals.at[0], deg_sh.at[didx.at[0]],
                         ssem[0], add=True).wait()

    plsc.subcore_barrier()
    pltpu.sync_copy(deg_sh.at[pl.ds(s * ROWS_PER_TILE, ROWS_PER_TILE)],
                    out_hbm.at[c, pl.ds(s * ROWS_PER_TILE, ROWS_PER_TILE)])


def _deg(mask, src, dst):
    f = pl.kernel(
        _deg_body,
        out_type=jax.ShapeDtypeStruct((NSC, NP), jnp.float32),
        mesh=_mesh(),
        scratch_types=[
            pltpu.VMEM((E_T_DEG,), jnp.int32),
            pltpu.VMEM((2, 128), jnp.int32),
            pltpu.VMEM((2, 128), jnp.float32),
            pltpu.VMEM((ROWS_PER_TILE,), jnp.float32),
            pltpu.VMEM_SHARED((NP,), jnp.float32),
            pltpu.VMEM_SHARED((NP,), jnp.float32),
            pltpu.SemaphoreType.DMA,
            pltpu.SemaphoreType.DMA,
            pltpu.SemaphoreType.DMA,
            pltpu.SemaphoreType.DMA,
            pltpu.SemaphoreType.DMA,
            pltpu.SemaphoreType.DMA,
        ],
    )
    return f(mask, src, dst)


E_T = 9984
NCH = 78
E_ROWS = N_EDGES // 128


def _seg_body(hp0, hp1, hp2, hp3, src_hbm, dst_hbm, o0, o1, o2, o3,
              sidx, sidx_x, didx, rows, zb, acc,
              gs0, gs1, ss0, ss1, ds0, ds1):
    c = lax.axis_index("c")
    s = lax.axis_index("s")
    gsem = (gs0, gs1)
    ssem = (ss0, ss1)
    dsem = (ds0, ds1)

    def zset(i, _):
        zb[i // 8, pl.ds((i % 8) * 16, 16)] = jnp.zeros((16,), jnp.float32)
        return 0
    lax.fori_loop(0, 16 * 8, zset, 0)

    pltpu.sync_copy(src_hbm.at[pl.ds(s * E_T, E_T)], sidx)

    @pl.when(s == 0)
    def _():
        pltpu.sync_copy(src_hbm.at[pl.ds(NTL * E_T, 256)], sidx_x)

    def one_block(hp_in, o_out):
        def zcp(j, _):
            pltpu.sync_copy(zb, acc.at[pl.ds(s * ROWS_PER_TILE + j * 16, 16), :])
            return 0
        lax.fori_loop(0, ROWS_PER_TILE // 16, zcp, 0)
        plsc.subcore_barrier()

        g_d = [None, None]
        s_d = [None, None]
        d_d = [None, None]
        d_d[0] = pltpu.async_copy(
            dst_hbm.at[pl.ds(s * E_T, 128)], didx.at[0], dsem[0])
        g_d[0] = pltpu.async_copy(
            hp_in.at[sidx.at[pl.ds(0, 128)]], rows.at[0], gsem[0])
        for i in range(NCH):
            b = i & 1
            bn = 1 - b
            if i + 1 < NCH:
                if s_d[bn] is not None:
                    s_d[bn].wait()
                d_d[bn] = pltpu.async_copy(
                    dst_hbm.at[pl.ds(s * E_T + (i + 1) * 128, 128)],
                    didx.at[bn], dsem[bn])
                g_d[bn] = pltpu.async_copy(
                    hp_in.at[sidx.at[pl.ds((i + 1) * 128, 128)]],
                    rows.at[bn], gsem[bn])
            g_d[b].wait()
            d_d[b].wait()
            s_d[b] = pltpu.async_copy(
                rows.at[b], acc.at[didx.at[b]], ssem[b], add=True)
        s_d[0].wait()
        s_d[1].wait()

        @pl.when(s == 0)
        def _():
            for j in range(2):
                pltpu.async_copy(dst_hbm.at[pl.ds(NTL * E_T + j * 128, 128)],
                                 didx.at[0], dsem[0]).wait()
                pltpu.async_copy(hp_in.at[sidx_x.at[pl.ds(j * 128, 128)]],
                                 rows.at[0], gsem[0]).wait()
                pltpu.async_copy(rows.at[0], acc.at[didx.at[0]],
                                 ssem[0], add=True).wait()

        plsc.subcore_barrier()
        pltpu.sync_copy(acc.at[pl.ds(s * ROWS_PER_TILE, ROWS_PER_TILE), :],
                        o_out.at[pl.ds(s * ROWS_PER_TILE, ROWS_PER_TILE), :])
        plsc.subcore_barrier()

    @pl.when(c == 0)
    def _():
        one_block(hp0, o0)
        one_block(hp1, o1)

    @pl.when(c == 1)
    def _():
        one_block(hp2, o2)
        one_block(hp3, o3)


def _seg(hps, src, dst):
    f = pl.kernel(
        _seg_body,
        out_type=tuple(jax.ShapeDtypeStruct((NP, 128), jnp.float32)
                       for _ in range(4)),
        mesh=_mesh(),
        scratch_types=[
            pltpu.VMEM((E_T,), jnp.int32),
            pltpu.VMEM((256,), jnp.int32),
            pltpu.VMEM((2, 128), jnp.int32),
            pltpu.VMEM((2, 128, 128), jnp.float32),
            pltpu.VMEM((16, 128), jnp.float32),
            pltpu.VMEM_SHARED((NP, 128), jnp.float32),
            pltpu.SemaphoreType.DMA,
            pltpu.SemaphoreType.DMA,
            pltpu.SemaphoreType.DMA,
            pltpu.SemaphoreType.DMA,
            pltpu.SemaphoreType.DMA,
            pltpu.SemaphoreType.DMA,
        ],
    )
    return f(hps[0], hps[1], hps[2], hps[3], src, dst)


def _mm_body1(xa_ref, sa_ref, w_ref, degt_ref, o0, o1, o2, o3):
    x = xa_ref[...] * sa_ref[...]
    h = jnp.dot(x, w_ref[...], preferred_element_type=jnp.float32)
    dinv = lax.rsqrt(degt_ref[...].sum(axis=1, keepdims=True) + 2.0)
    h = h * dinv
    o0[...] = h[:, 0:128]
    o1[...] = h[:, 128:256]
    o2[...] = h[:, 256:384]
    o3[...] = h[:, 384:512]


def _mm_body2(xa_ref, sa_ref, xb_ref, sb_ref, w_ref, degt_ref, o0, o1, o2, o3):
    x = xa_ref[...] * sa_ref[...] + xb_ref[...] * sb_ref[...]
    h = jnp.dot(x, w_ref[...], preferred_element_type=jnp.float32)
    dinv = lax.rsqrt(degt_ref[...].sum(axis=1, keepdims=True) + 2.0)
    h = h * dinv
    o0[...] = h[:, 0:128]
    o1[...] = h[:, 128:256]
    o2[...] = h[:, 256:384]
    o3[...] = h[:, 384:512]


def _mm_scale(xa, sa, W, degt, xb=None, sb=None):
    din = xa.shape[1]
    grid = (NP // RB,)
    row_spec = pl.BlockSpec((RB, din), lambda r: (r, 0))
    col_spec = pl.BlockSpec((RB, 1), lambda r: (r, 0))
    w_spec = pl.BlockSpec((din, HID), lambda r: (0, 0))
    deg_spec = pl.BlockSpec((RB, 2), lambda r: (r, 0))
    out_specs = tuple(pl.BlockSpec((RB, 128), lambda r: (r, 0)) for _ in range(4))
    out_shape = tuple(jax.ShapeDtypeStruct((NP, 128), jnp.float32) for _ in range(4))
    if xb is None:
        return pl.pallas_call(
            _mm_body1, grid=grid,
            in_specs=[row_spec, col_spec, w_spec, deg_spec],
            out_specs=out_specs, out_shape=out_shape,
        )(xa, sa, W, degt)
    return pl.pallas_call(
        _mm_body2, grid=grid,
        in_specs=[row_spec, col_spec, row_spec, col_spec, w_spec, deg_spec],
        out_specs=out_specs, out_shape=out_shape,
    )(xa, sa, xb, sb, W, degt)


def _comb_body(s0, s1, s2, s3, h0, h1, h2, h3, degt_ref, b_ref, o_ref, *, relu):
    dinv = lax.rsqrt(degt_ref[...].sum(axis=1, keepdims=True) + 2.0)
    parts = []
    for s_ref, h_ref in ((s0, h0), (s1, h1), (s2, h2), (s3, h3)):
        parts.append(dinv * (s_ref[...] + 2.0 * h_ref[...]))
    v = jnp.concatenate(parts, axis=1) + b_ref[...]
    if relu:
        v = jnp.maximum(v, 0.0)
    row = (pl.program_id(0) * RB
           + lax.broadcasted_iota(jnp.int32, (RB, 1), 0))
    o_ref[...] = jnp.where(row < N_NODES, v, 0.0)


def _combine(segs, hps, degt, b, relu):
    grid = (NP // RB,)
    blk = pl.BlockSpec((RB, 128), lambda r: (r, 0))
    deg_spec = pl.BlockSpec((RB, 2), lambda r: (r, 0))
    b_spec = pl.BlockSpec((1, HID), lambda r: (0, 0))
    return pl.pallas_call(
        functools.partial(_comb_body, relu=relu), grid=grid,
        in_specs=[blk] * 8 + [deg_spec, b_spec],
        out_specs=pl.BlockSpec((RB, HID), lambda r: (r, 0)),
        out_shape=jax.ShapeDtypeStruct((NP, HID), jnp.float32),
    )(*segs, *hps, degt, b.reshape(1, HID))


def _fused_comb(srefs, prefs, degt_ref, b_ref, relu=True):
    dinv = lax.rsqrt(degt_ref[...].sum(axis=1, keepdims=True) + 2.0)
    parts = [dinv * (s[...] + 2.0 * p[...]) for s, p in zip(srefs, prefs)]
    v = jnp.concatenate(parts, axis=1) + b_ref[...]
    return jnp.maximum(v, 0.0) if relu else v


def _mmf_body(ha_ref, ma_ref, s0, s1, s2, s3, p0, p1, p2, p3,
              degtb_ref, bb_ref, mb_ref, w_ref, degt_ref, o0, o1, o2, o3):
    xb = _fused_comb((s0, s1, s2, s3), (p0, p1, p2, p3), degtb_ref, bb_ref)
    x = ha_ref[...] * ma_ref[...] + xb * mb_ref[...]
    h = jnp.dot(x, w_ref[...], preferred_element_type=jnp.float32)
    dinv = lax.rsqrt(degt_ref[...].sum(axis=1, keepdims=True) + 2.0)
    h = h * dinv
    o0[...] = h[:, 0:128]
    o1[...] = h[:, 128:256]
    o2[...] = h[:, 256:384]
    o3[...] = h[:, 384:512]


def _mm_scale_fused(ha, ma, segs, hps, degtb, bb, mb, W, degt):
    grid = (NP // RB,)
    row_spec = pl.BlockSpec((RB, HID), lambda r: (r, 0))
    col_spec = pl.BlockSpec((RB, 1), lambda r: (r, 0))
    blk = pl.BlockSpec((RB, 128), lambda r: (r, 0))
    deg_spec = pl.BlockSpec((RB, 2), lambda r: (r, 0))
    b_spec = pl.BlockSpec((1, HID), lambda r: (0, 0))
    w_spec = pl.BlockSpec((HID, HID), lambda r: (0, 0))
    out_specs = tuple(pl.BlockSpec((RB, 128), lambda r: (r, 0)) for _ in range(4))
    out_shape = tuple(jax.ShapeDtypeStruct((NP, 128), jnp.float32) for _ in range(4))
    return pl.pallas_call(
        _mmf_body, grid=grid,
        in_specs=[row_spec, col_spec] + [blk] * 8
                 + [deg_spec, b_spec, col_spec, w_spec, deg_spec],
        out_specs=out_specs, out_shape=out_shape,
    )(ha, ma, *segs, *hps, degtb, bb.reshape(1, HID), mb, W, degt)


def _headf_body(s0, s1, s2, s3, p0, p1, p2, p3, degt_ref, bb_ref,
                batch_ref, w1_ref, b1_ref, w2_ref, b2_ref, o_ref):
    u = _fused_comb((s0, s1, s2, s3), (p0, p1, p2, p3), degt_ref, bb_ref,
                    relu=False)
    b_row = batch_ref[...]
    gid = lax.broadcasted_iota(jnp.int32, (N_GRAPHS, NP), 0)
    Bm = (b_row == gid).astype(jnp.float32)
    sums = jnp.dot(Bm, u, preferred_element_type=jnp.float32)
    cnt = jnp.sum(Bm, axis=1, keepdims=True)
    g = sums / jnp.maximum(cnt, 1.0)
    hdn = jnp.maximum(
        jnp.dot(g, w1_ref[...], preferred_element_type=jnp.float32)
        + b1_ref[...], 0.0)
    o_ref[...] = jnp.dot(hdn, w2_ref[...],
                         preferred_element_type=jnp.float32) + b2_ref[...]


def _head_fused(segs, hps, degt, bb, batch_p, Wm1, bm1, Wm2, bm2):
    return pl.pallas_call(
        _headf_body,
        out_shape=jax.ShapeDtypeStruct((N_GRAPHS, NOUT), jnp.float32),
    )(*segs, *hps, degt, bb.reshape(1, HID), batch_p,
      Wm1, bm1.reshape(1, NHID), Wm2, bm2.reshape(1, NOUT))


def _sortkey(score):
    u = lax.bitcast_convert_type(score, jnp.uint32)
    i = lax.bitcast_convert_type(score, jnp.int32)
    return jnp.where(i >= 0, u + jnp.uint32(0x80000000), ~u)


def _radix_max_ge(valid, keys, r, nbits):
    def body(i, t):
        b = nbits - 1 - i
        cand = t | (jnp.uint32(1) << jnp.uint32(b))
        cnt = jnp.sum(jnp.where(valid & (keys >= cand), 1.0, 0.0))
        return jnp.where(cnt >= r, cand, t)
    return lax.fori_loop(0, nbits, body, jnp.uint32(0))


def _pool_body(h3_ref, p_ref, mprev_ref, ks_ref, m_ref, sm_ref, key_ref, *, k):
    p = p_ref[...]
    norm = jnp.sqrt(jnp.sum(p * p))
    xp = jnp.sum(h3_ref[...] * p[None, :, :].reshape(1, 1, HID), axis=2)
    score = jnp.tanh(xp / norm)
    elig = mprev_ref[...] > 0.0
    ukey = jnp.where(elig, _sortkey(score), jnp.uint32(0))

    kf = jnp.float32(k)
    K = _radix_max_ge(elig, ukey, kf, 32)
    g = jnp.sum(jnp.where(ukey > K, 1.0, 0.0))
    r = kf - g
    tie = ukey == K

    ks = ks_ref[...]
    S = _radix_max_ge(tie, ks, r, 32)
    S = jnp.where(r > 0, S, jnp.uint32(0xFFFFFFFF))
    g2 = jnp.sum(jnp.where(tie & (ks > S), 1.0, 0.0))
    r2 = r - g2
    tie2 = tie & (ks == S)

    idx = (lax.broadcasted_iota(jnp.int32, (NPR, 128), 0) * 128
           + lax.broadcasted_iota(jnp.int32, (NPR, 128), 1))
    idxp = jnp.uint32(16383) - idx.astype(jnp.uint32)
    P = _radix_max_ge(tie2, idxp, r2, 14)
    P = jnp.where(r2 > 0, P, jnp.uint32(16384))

    sel = (ukey > K) | (tie & (ks > S)) | (tie2 & (idxp >= P))
    m = jnp.where(sel, 1.0, 0.0)
    m_ref[...] = m
    sm_ref[...] = score * m
    key_ref[...] = ukey


def _pool(h, p, mprev, ks, k):
    h3 = h.reshape(NPR, 128, HID)
    out = pl.pallas_call(
        functools.partial(_pool_body, k=k),
        out_shape=(
            jax.ShapeDtypeStruct((NPR, 128), jnp.float32),
            jax.ShapeDtypeStruct((NPR, 128), jnp.float32),
            jax.ShapeDtypeStruct((NPR, 128), jnp.uint32),
        ),
    )(h3, p.reshape(1, HID), mprev, ks)
    return out


def _head_body(batch_ref, u_ref, w1_ref, b1_ref, w2_ref, b2_ref, o_ref):
    b_row = batch_ref[...]
    gid = lax.broadcasted_iota(jnp.int32, (N_GRAPHS, NP), 0)
    Bm = (b_row == gid).astype(jnp.float32)
    sums = jnp.dot(Bm, u_ref[...], preferred_element_type=jnp.float32)
    cnt = jnp.sum(Bm, axis=1, keepdims=True)
    g = sums / jnp.maximum(cnt, 1.0)
    hdn = jnp.maximum(
        jnp.dot(g, w1_ref[...], preferred_element_type=jnp.float32)
        + b1_ref[...], 0.0)
    o_ref[...] = jnp.dot(hdn, w2_ref[...],
                         preferred_element_type=jnp.float32) + b2_ref[...]


def _head(batch_p, u0, Wm1, bm1, Wm2, bm2):
    return pl.pallas_call(
        _head_body,
        out_shape=jax.ShapeDtypeStruct((N_GRAPHS, NOUT), jnp.float32),
    )(batch_p, u0, Wm1, bm1.reshape(1, NHID), Wm2, bm2.reshape(1, NOUT))


def _pad_rows(a):
    return jnp.pad(a, ((0, NP - a.shape[0]), (0, 0)))


def kernel(x, edge_index, batch, Wd0, bd0, p1, Wd1, bd1, p2, Wd2, bd2,
           Wu0, bu0, Wu1, bu1, Wm1, bm1, Wm2, bm2):
    src = edge_index[0].astype(jnp.int32)
    dst = edge_index[1].astype(jnp.int32)
    xp = _pad_rows(x)
    ones_col = jnp.ones((NP, 1), jnp.float32)
    real = (jnp.arange(NP, dtype=jnp.int32) < N_NODES)
    ones_real = real.astype(jnp.float32)
    elig0 = ones_real.reshape(NPR, 128)
    zeros_key = jnp.zeros((NPR, 128), jnp.uint32)

    deg0t = _deg(ones_real, src, dst).T
    hp0 = _mm_scale(xp, ones_col, Wd0, deg0t)
    seg0 = _seg(hp0, src, dst)
    h0 = _combine(seg0, hp0, deg0t, bd0, relu=True)

    m1, sm1, key1 = _pool(h0, p1, elig0, zeros_key, K1)
    m1f = m1.reshape(NP)
    deg1t = _deg(m1f, src, dst).T
    hp1 = _mm_scale(h0, sm1.reshape(NP, 1), Wd1, deg1t)
    seg1 = _seg(hp1, src, dst)
    h1 = _combine(seg1, hp1, deg1t, bd1, relu=True)

    m2, sm2, key2 = _pool(h1, p2, m1, key1, K2)
    m2f = m2.reshape(NP)
    deg2t = _deg(m2f, src, dst).T
    hp2 = _mm_scale(h1, sm2.reshape(NP, 1), Wd2, deg2t)
    seg2 = _seg(hp2, src, dst)

    hpu1 = _mm_scale_fused(h1, m1.reshape(NP, 1), seg2, hp2, deg2t, bd2,
                           m2.reshape(NP, 1), Wu0, deg1t)
    segu1 = _seg(hpu1, src, dst)

    hpu0 = _mm_scale_fused(h0, ones_col, segu1, hpu1, deg1t, bu0,
                           m1.reshape(NP, 1), Wu1, deg0t)
    segu0 = _seg(hpu0, src, dst)

    batch_p = jnp.pad(batch.astype(jnp.int32), (0, NP - N_NODES),
                      constant_values=N_GRAPHS).reshape(1, NP)
    return _head_fused(segu0, hpu0, deg0t, bu1, batch_p, Wm1, bm1, Wm2, bm2)
